# baseline (speedup 1.0000x reference)
import jax
import jax.numpy as jnp
from jax import lax
from jax.experimental import pallas as pl
from jax.experimental.pallas import tpu as pltpu
from jax.experimental.pallas import tpu_sc as plsc

B = 16384
D = 32
NT = 100000
_NC, _NS = 2, 16
_NW = _NC * _NS
_BPW = B // _NW
_CH = 128
_NCH = _BPW // _CH
_IDX_ROWS = B // _CH


def _sc_body(aidx_h, tidx_h, iaidx_h, itidx_h, aeu_h, teu_h, aei_h, tei_h,
             wa_h, wb_h,
             aue_o, tue_o, aie_o, tie_o, waa_o, wat_o, wba_o, wbt_o,
             aidx, tidx, aiidx, tiidx,
             g0, g1, g2, g3, g4, g5, g6, g7, sem):
    wid = lax.axis_index("s") * _NC + lax.axis_index("c")
    rbase = wid * _BPW
    ibase = wid * _NCH
    pltpu.sync_copy(aidx_h.at[pl.ds(ibase, _NCH)], aidx)
    pltpu.sync_copy(tidx_h.at[pl.ds(ibase, _NCH)], tidx)
    pltpu.sync_copy(iaidx_h.at[pl.ds(ibase, _NCH)], aiidx)
    pltpu.sync_copy(itidx_h.at[pl.ds(ibase, _NCH)], tiidx)

    copies = []
    for c in range(_NCH):
        s = pl.ds(c * _CH, _CH)
        copies.append(pltpu.async_copy(aeu_h.at[aidx.at[c]], g0.at[s], sem))
        copies.append(pltpu.async_copy(teu_h.at[tidx.at[c]], g1.at[s], sem))
        copies.append(pltpu.async_copy(aei_h.at[aiidx.at[c]], g2.at[s], sem))
        copies.append(pltpu.async_copy(tei_h.at[tiidx.at[c]], g3.at[s], sem))
        copies.append(pltpu.async_copy(wa_h.at[aidx.at[c]], g4.at[s], sem))
        copies.append(pltpu.async_copy(wa_h.at[tidx.at[c]], g5.at[s], sem))
        copies.append(pltpu.async_copy(wb_h.at[aidx.at[c]], g6.at[s], sem))
        copies.append(pltpu.async_copy(wb_h.at[tidx.at[c]], g7.at[s], sem))
    for d in copies:
        d.wait()
    out_slice = pl.ds(rbase, _BPW)
    pltpu.sync_copy(g0, aue_o.at[out_slice])
    pltpu.sync_copy(g1, tue_o.at[out_slice])
    pltpu.sync_copy(g2, aie_o.at[out_slice])
    pltpu.sync_copy(g3, tie_o.at[out_slice])
    pltpu.sync_copy(g4, waa_o.at[out_slice])
    pltpu.sync_copy(g5, wat_o.at[out_slice])
    pltpu.sync_copy(g6, wba_o.at[out_slice])
    pltpu.sync_copy(g7, wbt_o.at[out_slice])


_sc_gather = pl.kernel(
    _sc_body,
    out_type=[jax.ShapeDtypeStruct((B, D), jnp.bfloat16)] * 8,
    mesh=plsc.VectorSubcoreMesh(core_axis_name="c", subcore_axis_name="s"),
    compiler_params=pltpu.CompilerParams(use_tc_tiling_on_sc=False),
    scratch_types=(
        [pltpu.VMEM((_NCH, _CH), jnp.int32)] * 4
        + [pltpu.VMEM((_BPW, D), jnp.bfloat16)] * 8
        + [pltpu.SemaphoreType.DMA]
    ),
)

_BT = 512
_GRID = (B // 4) // _BT
_PK = B // 4


def _tc_body(ar_ref, tr_ref,
             aue, tue, aie, tie, waa, wat, wba, wbt,
             w1au, b1au, w2au, b2au,
             w1tu, b1tu, w2tu, b2tu,
             w1ai, b1ai, w2ai, b2ai,
             w1ti, b1ti, w2ti, b2ti,
             sel, la_ref, lt_ref):
    i = pl.program_id(0)
    a_e = aue[...].astype(jnp.float32)
    t_e = tue[...].astype(jnp.float32)
    x_au = waa[...].astype(jnp.float32) * a_e + \
        (1.0 - wat[...].astype(jnp.float32)) * t_e
    x_tu = wba[...].astype(jnp.float32) * a_e + \
        (1.0 - wbt[...].astype(jnp.float32)) * t_e
    x_ai = aie[...].astype(jnp.float32)
    x_ti = tie[...].astype(jnp.float32)

    def mlp(x, w1, b1, w2, b2):
        h = jnp.maximum(
            jnp.dot(x, w1[...], preferred_element_type=jnp.float32)
            + b1[...], 0.0)
        return jnp.maximum(
            jnp.dot(h, w2[...], preferred_element_type=jnp.float32)
            + b2[...], 0.0)

    y_au = mlp(x_au, w1au, b1au, w2au, b2au)
    y_tu = mlp(x_tu, w1tu, b1tu, w2tu, b2tu)
    y_ai = mlp(x_ai, w1ai, b1ai, w2ai, b2ai)
    y_ti = mlp(x_ti, w1ti, b1ti, w2ti, b2ti)

    s_a = jnp.maximum(
        jnp.dot(y_au * y_ai, sel[...], preferred_element_type=jnp.float32),
        1e-6)
    s_t = jnp.maximum(
        jnp.dot(y_tu * y_ti, sel[...], preferred_element_type=jnp.float32),
        1e-6)
    da = s_a - ar_ref[...]
    dt = s_t - tr_ref[...]
    pa = jnp.sum(da * da) * (1.0 / B)
    pt = jnp.sum(dt * dt) * (1.0 / B)

    @pl.when(i == 0)
    def _():
        la_ref[0, 0] = 0.0
        lt_ref[0, 0] = 0.0

    la_ref[0, 0] += pa
    lt_ref[0, 0] += pt


def _wspec():
    return pl.BlockSpec((4 * D, 8 * D), lambda i: (0, 0))


def _bspec():
    return pl.BlockSpec((1, 8 * D), lambda i: (0, 0))


def _w2spec():
    return pl.BlockSpec((8 * D, 4 * D), lambda i: (0, 0))


def _b2spec():
    return pl.BlockSpec((1, 4 * D), lambda i: (0, 0))


_tc_dense = pl.pallas_call(
    _tc_body,
    grid=(_GRID,),
    in_specs=[
        pl.BlockSpec((_BT, 4), lambda i: (i, 0)),
        pl.BlockSpec((_BT, 4), lambda i: (i, 0)),
    ] + [pl.BlockSpec((_BT, 4 * D), lambda i: (i, 0))] * 8 + [
        _wspec(), _bspec(), _w2spec(), _b2spec(),
        _wspec(), _bspec(), _w2spec(), _b2spec(),
        _wspec(), _bspec(), _w2spec(), _b2spec(),
        _wspec(), _bspec(), _w2spec(), _b2spec(),
        pl.BlockSpec((4 * D, 4), lambda i: (0, 0)),
    ],
    out_specs=[
        pl.BlockSpec(memory_space=pltpu.SMEM),
        pl.BlockSpec(memory_space=pltpu.SMEM),
    ],
    out_shape=[jax.ShapeDtypeStruct((1, 1), jnp.float32)] * 2,
)


def _block_diag4(w):
    d_in, d_out = w.shape
    full = jnp.zeros((4 * d_in, 4 * d_out), dtype=jnp.float32)
    for i in range(4):
        full = full.at[i * d_in:(i + 1) * d_in, i * d_out:(i + 1) * d_out].set(w)
    return full


def kernel(ausers, aitems, aratings, tusers, titems, tratings, params):
    p = params
    au2 = ausers.reshape(_IDX_ROWS, _CH)
    tu2 = tusers.reshape(_IDX_ROWS, _CH)
    ai2 = aitems.reshape(_IDX_ROWS, _CH)
    ti2 = titems.reshape(_IDX_ROWS, _CH)
    gathered = _sc_gather(
        au2, tu2, ai2, ti2,
        p["a_emb_user"].astype(jnp.bfloat16),
        p["t_emb_user"].astype(jnp.bfloat16),
        p["a_emb_item"].astype(jnp.bfloat16),
        p["t_emb_item"].astype(jnp.bfloat16),
        p["W_a"].astype(jnp.bfloat16),
        p["W_b"].astype(jnp.bfloat16))
    packed = [g.reshape(_PK, 4 * D) for g in gathered]

    wargs = []
    for name in ("mlp_a_users", "mlp_t_users", "mlp_a_items", "mlp_t_items"):
        m = p[name]
        wargs += [
            _block_diag4(m["W1"]),
            jnp.tile(m["b1"], 4).reshape(1, 8 * D),
            _block_diag4(m["W2"]),
            jnp.tile(m["b2"], 4).reshape(1, 4 * D),
        ]
    sel = (jnp.arange(4 * D)[:, None] // D ==
           jnp.arange(4)[None, :]).astype(jnp.float32)

    ar2 = aratings.astype(jnp.float32).reshape(_PK, 4)
    tr2 = tratings.astype(jnp.float32).reshape(_PK, 4)
    la, lt = _tc_dense(ar2, tr2, *packed, *wargs, sel)
    return (la[0, 0], lt[0, 0])


# speedup vs baseline: 1.4982x; 1.4982x over previous
import jax
import jax.numpy as jnp
from jax import lax
from jax.experimental import pallas as pl
from jax.experimental.pallas import tpu as pltpu
from jax.experimental.pallas import tpu_sc as plsc

B = 16384
D = 32
NT = 100000
_NC, _NS = 2, 16
_NW = _NC * _NS
_BPW = B // _NW
_CH = 128
_NCH = _BPW // _CH
_IDX_ROWS = B // _CH


def _sc_body(aidx_h, tidx_h, iaidx_h, itidx_h, aeu_h, teu_h, aei_h, tei_h,
             wa_h, wb_h,
             aue_o, tue_o, aie_o, tie_o, waa_o, wat_o, wba_o, wbt_o,
             aidx, tidx, aiidx, tiidx,
             g0, g1, g2, g3, g4, g5, g6, g7,
             h0, h1, h2, h3, h4, h5, h6, h7, sem):
    wid = lax.axis_index("s") * _NC + lax.axis_index("c")
    rbase = wid * _BPW
    ibase = wid * _NCH
    pltpu.sync_copy(aidx_h.at[pl.ds(ibase, _NCH)], aidx)
    pltpu.sync_copy(tidx_h.at[pl.ds(ibase, _NCH)], tidx)
    pltpu.sync_copy(iaidx_h.at[pl.ds(ibase, _NCH)], aiidx)
    pltpu.sync_copy(itidx_h.at[pl.ds(ibase, _NCH)], tiidx)

    srcs = [(aeu_h, aidx), (teu_h, tidx), (aei_h, aiidx), (tei_h, tiidx),
            (wa_h, aidx), (wa_h, tidx), (wb_h, aidx), (wb_h, tidx)]
    outs = [aue_o, tue_o, aie_o, tie_o, waa_o, wat_o, wba_o, wbt_o]
    bufs = [[g0, g1], [g2, g3], [g4, g5], [g6, g7],
            [h0, h1], [h2, h3], [h4, h5], [h6, h7]]

    def fire(c):
        return [pltpu.async_copy(tbl.at[idx.at[c]], bufs[a][c % 2], sem)
                for a, (tbl, idx) in enumerate(srcs)]

    def copy_out(c):
        for a in range(8):
            pltpu.sync_copy(bufs[a][c % 2],
                            outs[a].at[pl.ds(rbase + c * _CH, _CH)])

    prev = fire(0)
    for c in range(1, _NCH):
        cur = fire(c)
        for d in prev:
            d.wait()
        copy_out(c - 1)
        prev = cur
    for d in prev:
        d.wait()
    copy_out(_NCH - 1)


_sc_gather = pl.kernel(
    _sc_body,
    out_type=[jax.ShapeDtypeStruct((B, D), jnp.float32)] * 8,
    mesh=plsc.VectorSubcoreMesh(core_axis_name="c", subcore_axis_name="s"),
    compiler_params=pltpu.CompilerParams(use_tc_tiling_on_sc=False),
    scratch_types=(
        [pltpu.VMEM((_NCH, _CH), jnp.int32)] * 4
        + [pltpu.VMEM((_CH, D), jnp.float32)] * 16
        + [pltpu.SemaphoreType.DMA]
    ),
)

_BT = 512
_GRID = (B // 4) // _BT
_PK = B // 4


def _tc_body(ar_ref, tr_ref,
             aue, tue, aie, tie, waa, wat, wba, wbt,
             w1au, b1au, w2au, b2au,
             w1tu, b1tu, w2tu, b2tu,
             w1ai, b1ai, w2ai, b2ai,
             w1ti, b1ti, w2ti, b2ti,
             sel, la_ref, lt_ref):
    i = pl.program_id(0)
    a_e = aue[...].astype(jnp.float32)
    t_e = tue[...].astype(jnp.float32)
    x_au = waa[...].astype(jnp.float32) * a_e + \
        (1.0 - wat[...].astype(jnp.float32)) * t_e
    x_tu = wba[...].astype(jnp.float32) * a_e + \
        (1.0 - wbt[...].astype(jnp.float32)) * t_e
    x_ai = aie[...].astype(jnp.float32)
    x_ti = tie[...].astype(jnp.float32)

    def mlp(x, w1, b1, w2, b2):
        h = jnp.maximum(
            jnp.dot(x, w1[...], preferred_element_type=jnp.float32)
            + b1[...], 0.0)
        return jnp.maximum(
            jnp.dot(h, w2[...], preferred_element_type=jnp.float32)
            + b2[...], 0.0)

    y_au = mlp(x_au, w1au, b1au, w2au, b2au)
    y_tu = mlp(x_tu, w1tu, b1tu, w2tu, b2tu)
    y_ai = mlp(x_ai, w1ai, b1ai, w2ai, b2ai)
    y_ti = mlp(x_ti, w1ti, b1ti, w2ti, b2ti)

    s_a = jnp.maximum(
        jnp.dot(y_au * y_ai, sel[...], preferred_element_type=jnp.float32),
        1e-6)
    s_t = jnp.maximum(
        jnp.dot(y_tu * y_ti, sel[...], preferred_element_type=jnp.float32),
        1e-6)
    da = s_a - ar_ref[...]
    dt = s_t - tr_ref[...]
    pa = jnp.sum(da * da) * (1.0 / B)
    pt = jnp.sum(dt * dt) * (1.0 / B)

    @pl.when(i == 0)
    def _():
        la_ref[0, 0] = 0.0
        lt_ref[0, 0] = 0.0

    la_ref[0, 0] += pa
    lt_ref[0, 0] += pt


def _wspec():
    return pl.BlockSpec((4 * D, 8 * D), lambda i: (0, 0))


def _bspec():
    return pl.BlockSpec((1, 8 * D), lambda i: (0, 0))


def _w2spec():
    return pl.BlockSpec((8 * D, 4 * D), lambda i: (0, 0))


def _b2spec():
    return pl.BlockSpec((1, 4 * D), lambda i: (0, 0))


_tc_dense = pl.pallas_call(
    _tc_body,
    grid=(_GRID,),
    in_specs=[
        pl.BlockSpec((_BT, 4), lambda i: (i, 0)),
        pl.BlockSpec((_BT, 4), lambda i: (i, 0)),
    ] + [pl.BlockSpec((_BT, 4 * D), lambda i: (i, 0))] * 8 + [
        _wspec(), _bspec(), _w2spec(), _b2spec(),
        _wspec(), _bspec(), _w2spec(), _b2spec(),
        _wspec(), _bspec(), _w2spec(), _b2spec(),
        _wspec(), _bspec(), _w2spec(), _b2spec(),
        pl.BlockSpec((4 * D, 4), lambda i: (0, 0)),
    ],
    out_specs=[
        pl.BlockSpec(memory_space=pltpu.SMEM),
        pl.BlockSpec(memory_space=pltpu.SMEM),
    ],
    out_shape=[jax.ShapeDtypeStruct((1, 1), jnp.float32)] * 2,
)


def _block_diag4(w):
    d_in, d_out = w.shape
    full = jnp.zeros((4 * d_in, 4 * d_out), dtype=jnp.float32)
    for i in range(4):
        full = full.at[i * d_in:(i + 1) * d_in, i * d_out:(i + 1) * d_out].set(w)
    return full


def kernel(ausers, aitems, aratings, tusers, titems, tratings, params):
    p = params
    au2 = ausers.reshape(_IDX_ROWS, _CH)
    tu2 = tusers.reshape(_IDX_ROWS, _CH)
    ai2 = aitems.reshape(_IDX_ROWS, _CH)
    ti2 = titems.reshape(_IDX_ROWS, _CH)
    gathered = _sc_gather(
        au2, tu2, ai2, ti2,
        p["a_emb_user"], p["t_emb_user"], p["a_emb_item"], p["t_emb_item"],
        p["W_a"], p["W_b"])
    packed = [g.reshape(_PK, 4 * D) for g in gathered]

    wargs = []
    for name in ("mlp_a_users", "mlp_t_users", "mlp_a_items", "mlp_t_items"):
        m = p[name]
        wargs += [
            _block_diag4(m["W1"]),
            jnp.tile(m["b1"], 4).reshape(1, 8 * D),
            _block_diag4(m["W2"]),
            jnp.tile(m["b2"], 4).reshape(1, 4 * D),
        ]
    sel = (jnp.arange(4 * D)[:, None] // D ==
           jnp.arange(4)[None, :]).astype(jnp.float32)

    ar2 = aratings.astype(jnp.float32).reshape(_PK, 4)
    tr2 = tratings.astype(jnp.float32).reshape(_PK, 4)
    la, lt = _tc_dense(ar2, tr2, *packed, *wargs, sel)
    return (la[0, 0], lt[0, 0])


# speedup vs baseline: 1.5956x; 1.0650x over previous
import jax
import jax.numpy as jnp
from jax import lax
from jax.experimental import pallas as pl
from jax.experimental.pallas import tpu as pltpu
from jax.experimental.pallas import tpu_sc as plsc

B = 16384
D = 32
NT = 100000
_NC, _NS = 2, 16
_NW = _NC * _NS
_BPW = B // _NW
_CH = 128
_NCH = _BPW // _CH
_IDX_ROWS = B // _CH


def _sc_body(aidx_h, tidx_h, iaidx_h, itidx_h, aeu_h, teu_h, aei_h, tei_h,
             wa_h, wb_h,
             aue_o, tue_o, aie_o, tie_o, waa_o, wat_o, wba_o, wbt_o,
             aidx, tidx, aiidx, tiidx,
             g0, g1, g2, g3, g4, g5, g6, g7,
             h0, h1, h2, h3, h4, h5, h6, h7, sem):
    wid = lax.axis_index("s") * _NC + lax.axis_index("c")
    rbase = wid * _BPW
    ibase = wid * _NCH
    pltpu.sync_copy(aidx_h.at[pl.ds(ibase, _NCH)], aidx)
    pltpu.sync_copy(tidx_h.at[pl.ds(ibase, _NCH)], tidx)
    pltpu.sync_copy(iaidx_h.at[pl.ds(ibase, _NCH)], aiidx)
    pltpu.sync_copy(itidx_h.at[pl.ds(ibase, _NCH)], tiidx)

    srcs = [(aeu_h, aidx), (teu_h, tidx), (aei_h, aiidx), (tei_h, tiidx),
            (wa_h, aidx), (wa_h, tidx), (wb_h, aidx), (wb_h, tidx)]
    outs = [aue_o, tue_o, aie_o, tie_o, waa_o, wat_o, wba_o, wbt_o]
    bufs = [[g0, g1], [g2, g3], [g4, g5], [g6, g7],
            [h0, h1], [h2, h3], [h4, h5], [h6, h7]]

    def fire(c):
        return [pltpu.async_copy(tbl.at[idx.at[c]], bufs[a][c % 2], sem)
                for a, (tbl, idx) in enumerate(srcs)]

    def copy_out(c):
        for a in range(8):
            pltpu.sync_copy(bufs[a][c % 2],
                            outs[a].at[pl.ds(rbase + c * _CH, _CH)])

    prev = fire(0)
    for c in range(1, _NCH):
        cur = fire(c)
        for d in prev:
            d.wait()
        copy_out(c - 1)
        prev = cur
    for d in prev:
        d.wait()
    copy_out(_NCH - 1)


_sc_gather = pl.kernel(
    _sc_body,
    out_type=[jax.ShapeDtypeStruct((B, D), jnp.float32)] * 8,
    mesh=plsc.VectorSubcoreMesh(core_axis_name="c", subcore_axis_name="s"),
    compiler_params=pltpu.CompilerParams(use_tc_tiling_on_sc=False),
    scratch_types=(
        [pltpu.VMEM((_NCH, _CH), jnp.int32)] * 4
        + [pltpu.VMEM((_CH, D), jnp.float32)] * 16
        + [pltpu.SemaphoreType.DMA]
    ),
)

_BT = 512
_GRID = (B // 4) // _BT
_PK = B // 4


def _tc_body(ar_ref, tr_ref,
             aue, tue, aie, tie, waa, wat, wba, wbt,
             w1au, b1au, w2au, b2au,
             w1tu, b1tu, w2tu, b2tu,
             w1ai, b1ai, w2ai, b2ai,
             w1ti, b1ti, w2ti, b2ti,
             sel, la_ref, lt_ref,
             w1s0, w1s1, w1s2, w1s3, w2s0, w2s1, w2s2, w2s3):
    i = pl.program_id(0)

    @pl.when(i == 0)
    def _():
        for ws, w, d_in, d_out in (
                (w1s0, w1au, D, 2 * D), (w1s1, w1tu, D, 2 * D),
                (w1s2, w1ai, D, 2 * D), (w1s3, w1ti, D, 2 * D),
                (w2s0, w2au, 2 * D, D), (w2s1, w2tu, 2 * D, D),
                (w2s2, w2ai, 2 * D, D), (w2s3, w2ti, 2 * D, D)):
            ws[...] = jnp.zeros((4 * d_in, 4 * d_out), jnp.float32)
            for k in range(4):
                ws[k * d_in:(k + 1) * d_in, k * d_out:(k + 1) * d_out] = w[...]
    a_e = aue[...].astype(jnp.float32)
    t_e = tue[...].astype(jnp.float32)
    x_au = waa[...].astype(jnp.float32) * a_e + \
        (1.0 - wat[...].astype(jnp.float32)) * t_e
    x_tu = wba[...].astype(jnp.float32) * a_e + \
        (1.0 - wbt[...].astype(jnp.float32)) * t_e
    x_ai = aie[...].astype(jnp.float32)
    x_ti = tie[...].astype(jnp.float32)

    def mlp(x, w1, b1, w2, b2):
        b1t = jnp.concatenate([b1[...]] * 4, axis=1)
        b2t = jnp.concatenate([b2[...]] * 4, axis=1)
        h = jnp.maximum(
            jnp.dot(x, w1[...], preferred_element_type=jnp.float32)
            + b1t, 0.0)
        return jnp.maximum(
            jnp.dot(h, w2[...], preferred_element_type=jnp.float32)
            + b2t, 0.0)

    y_au = mlp(x_au, w1s0, b1au, w2s0, b2au)
    y_tu = mlp(x_tu, w1s1, b1tu, w2s1, b2tu)
    y_ai = mlp(x_ai, w1s2, b1ai, w2s2, b2ai)
    y_ti = mlp(x_ti, w1s3, b1ti, w2s3, b2ti)

    s_a = jnp.maximum(
        jnp.dot(y_au * y_ai, sel[...], preferred_element_type=jnp.float32),
        1e-6)
    s_t = jnp.maximum(
        jnp.dot(y_tu * y_ti, sel[...], preferred_element_type=jnp.float32),
        1e-6)
    da = s_a - ar_ref[...]
    dt = s_t - tr_ref[...]
    pa = jnp.sum(da * da) * (1.0 / B)
    pt = jnp.sum(dt * dt) * (1.0 / B)

    @pl.when(i == 0)
    def _():
        la_ref[0, 0] = 0.0
        lt_ref[0, 0] = 0.0

    la_ref[0, 0] += pa
    lt_ref[0, 0] += pt


def _wspec():
    return pl.BlockSpec((D, 2 * D), lambda i: (0, 0))


def _bspec():
    return pl.BlockSpec((1, 2 * D), lambda i: (0, 0))


def _w2spec():
    return pl.BlockSpec((2 * D, D), lambda i: (0, 0))


def _b2spec():
    return pl.BlockSpec((1, D), lambda i: (0, 0))


_tc_dense = pl.pallas_call(
    _tc_body,
    grid=(_GRID,),
    in_specs=[
        pl.BlockSpec((_BT, 4), lambda i: (i, 0)),
        pl.BlockSpec((_BT, 4), lambda i: (i, 0)),
    ] + [pl.BlockSpec((_BT, 4 * D), lambda i: (i, 0))] * 8 + [
        _wspec(), _bspec(), _w2spec(), _b2spec(),
        _wspec(), _bspec(), _w2spec(), _b2spec(),
        _wspec(), _bspec(), _w2spec(), _b2spec(),
        _wspec(), _bspec(), _w2spec(), _b2spec(),
        pl.BlockSpec((4 * D, 4), lambda i: (0, 0)),
    ],
    out_specs=[
        pl.BlockSpec(memory_space=pltpu.SMEM),
        pl.BlockSpec(memory_space=pltpu.SMEM),
    ],
    out_shape=[jax.ShapeDtypeStruct((1, 1), jnp.float32)] * 2,
    scratch_shapes=(
        [pltpu.VMEM((4 * D, 8 * D), jnp.float32)] * 4
        + [pltpu.VMEM((8 * D, 4 * D), jnp.float32)] * 4
    ),
)


def kernel(ausers, aitems, aratings, tusers, titems, tratings, params):
    p = params
    au2 = ausers.reshape(_IDX_ROWS, _CH)
    tu2 = tusers.reshape(_IDX_ROWS, _CH)
    ai2 = aitems.reshape(_IDX_ROWS, _CH)
    ti2 = titems.reshape(_IDX_ROWS, _CH)
    gathered = _sc_gather(
        au2, tu2, ai2, ti2,
        p["a_emb_user"], p["t_emb_user"], p["a_emb_item"], p["t_emb_item"],
        p["W_a"], p["W_b"])
    packed = [g.reshape(_PK, 4 * D) for g in gathered]

    wargs = []
    for name in ("mlp_a_users", "mlp_t_users", "mlp_a_items", "mlp_t_items"):
        m = p[name]
        wargs += [
            m["W1"],
            m["b1"].reshape(1, 2 * D),
            m["W2"],
            m["b2"].reshape(1, D),
        ]
    sel = (jnp.arange(4 * D)[:, None] // D ==
           jnp.arange(4)[None, :]).astype(jnp.float32)

    ar2 = aratings.astype(jnp.float32).reshape(_PK, 4)
    tr2 = tratings.astype(jnp.float32).reshape(_PK, 4)
    la, lt = _tc_dense(ar2, tr2, *packed, *wargs, sel)
    return (la[0, 0], lt[0, 0])
